# baseline (speedup 1.0000x reference)
import functools

import jax
import jax.numpy as jnp
from jax import lax
from jax.experimental import pallas as pl
from jax.experimental.pallas import tpu as pltpu
from jax.experimental.pallas import tpu_sc as plsc


def _make_sc_kernel(B, F, D, NW, CS):
    b_per_w = B // NW
    n_idx = b_per_w * F
    n_chunks = n_idx // CS

    mesh = plsc.VectorSubcoreMesh(core_axis_name="c", subcore_axis_name="s")

    @functools.partial(
        pl.kernel,
        mesh=mesh,
        compiler_params=pltpu.CompilerParams(use_tc_tiling_on_sc=False),
        out_type=jax.ShapeDtypeStruct((B, D), jnp.float32),
        scratch_types=[
            pltpu.VMEM((n_chunks, CS), jnp.int32),
            pltpu.VMEM((n_idx, D), jnp.float32),
            pltpu.VMEM((b_per_w, D), jnp.float32),
            pltpu.SemaphoreType.DMA,
        ],
    )
    def run(idx_hbm, table_hbm, out_hbm, idx_v, rows_v, out_v, sem):
        wid = lax.axis_index("s") * 2 + lax.axis_index("c")
        base = wid * b_per_w
        pltpu.sync_copy(idx_hbm.at[wid], idx_v)
        copies = []
        for j in range(n_chunks):
            copies.append(
                pltpu.async_copy(
                    table_hbm.at[idx_v.at[j]],
                    rows_v.at[pl.ds(j * CS, CS)],
                    sem,
                )
            )
        for c in copies:
            c.wait()

        def body(b, _):
            p = b * F
            acc = rows_v[p, :]
            for f in range(1, F):
                acc = acc + rows_v[p + f, :]
            out_v[b, :] = acc
            return 0

        lax.fori_loop(0, b_per_w, body, 0)
        pltpu.sync_copy(out_v, out_hbm.at[pl.ds(base, b_per_w)])

    return run


def kernel(sparse_input, emb_table):
    B, F = sparse_input.shape
    V, D = emb_table.shape
    NW = 32
    CS = 128
    b_per_w = B // NW
    n_chunks = (b_per_w * F) // CS
    idx = sparse_input.reshape(NW, n_chunks, CS)
    run = _make_sc_kernel(B, F, D, NW, CS)
    return run(idx, emb_table)
